# Initial kernel scaffold; baseline (speedup 1.0000x reference)
import jax
import jax.numpy as jnp
from jax.experimental import pallas as pl


def kernel(hidden_states, router_w, e_bias, expert_up, expert_down, shared_up, shared_down):
    raise NotImplementedError("write your pallas kernel here")


# speedup vs baseline: 2.2425x; 2.2425x over previous
import functools

import jax
import jax.numpy as jnp
from jax.experimental import pallas as pl

N_EXPERTS = 8
TOP_K = 2
N_GROUP = 4
HIDDEN = 1024
MOE_FF = 512
SHARED_FF = 1024

BT = 512

_NEG = -1e30


def _row_argmax(m, lane):
    v = jnp.max(m, axis=1, keepdims=True)
    idx = jnp.min(jnp.where(m == v, lane, 127), axis=1, keepdims=True)
    return v, idx


def _router_block(x, rw, eb):
    logits = jax.lax.dot_general(
        x.astype(jnp.bfloat16), rw.astype(jnp.bfloat16),
        (((1,), (1,)), ((), ())), preferred_element_type=jnp.float32)
    scores = 1.0 / (1.0 + jnp.exp(-logits))
    sfc = scores + eb
    lane = jax.lax.broadcasted_iota(jnp.int32, (BT, N_EXPERTS), 1)
    gid = lane // (N_EXPERTS // N_GROUP)
    gsum = jnp.zeros_like(sfc)
    for g in range(N_GROUP):
        gs = jnp.sum(jnp.where(gid == g, sfc, 0.0), axis=1, keepdims=True)
        gsum = jnp.where(gid == g, gs, gsum)
    g1v = jnp.max(gsum, axis=1, keepdims=True)
    g1 = jnp.min(jnp.where(gsum == g1v, gid, 127), axis=1, keepdims=True)
    gsum2 = jnp.where(gid == g1, _NEG, gsum)
    g2v = jnp.max(gsum2, axis=1, keepdims=True)
    g2 = jnp.min(jnp.where(gsum2 == g2v, gid, 127), axis=1, keepdims=True)
    sel = (gid == g1) | (gid == g2)
    m = jnp.where(sel, sfc, 0.0)
    _, i1 = _row_argmax(m, lane)
    m2 = jnp.where(lane == i1, _NEG, m)
    _, i2 = _row_argmax(m2, lane)
    w1 = jnp.sum(jnp.where(lane == i1, scores, 0.0), axis=1, keepdims=True)
    w2 = jnp.sum(jnp.where(lane == i2, scores, 0.0), axis=1, keepdims=True)
    denom = w1 + w2 + 1e-20
    w1 = w1 / denom
    w2 = w2 / denom
    return jnp.where(lane == i1, w1, 0.0) + jnp.where(lane == i2, w2, 0.0)


def _moe_body(x_ref, rw_ref, eb_ref, up_ref, dn_ref, su_ref, sd_ref, o_ref):
    x = x_ref[...]
    w_full = _router_block(x, rw_ref[...], eb_ref[...])
    h = jax.lax.dot_general(
        x, su_ref[...], (((1,), (1,)), ((), ())), preferred_element_type=jnp.float32)
    h = h * jax.nn.sigmoid(h)
    acc = jax.lax.dot_general(
        h, sd_ref[...], (((1,), (1,)), ((), ())), preferred_element_type=jnp.float32)
    for e in range(N_EXPERTS):
        he = jax.lax.dot_general(
            x, up_ref[e], (((1,), (1,)), ((), ())), preferred_element_type=jnp.float32)
        he = he * jax.nn.sigmoid(he)
        ye = jax.lax.dot_general(
            he, dn_ref[e], (((1,), (1,)), ((), ())), preferred_element_type=jnp.float32)
        acc = acc + w_full[:, e:e + 1] * ye
    o_ref[...] = acc


@jax.jit
def _moe(hs2, router_w, e_bias2, expert_up, expert_down, shared_up, shared_down):
    t = hs2.shape[0]
    grid = (t // BT,)
    const = lambda *shape: pl.BlockSpec(shape, lambda i: (0,) * len(shape))
    return pl.pallas_call(
        _moe_body,
        grid=grid,
        in_specs=[
            pl.BlockSpec((BT, HIDDEN), lambda i: (i, 0)),
            const(N_EXPERTS, HIDDEN),
            const(1, N_EXPERTS),
            const(N_EXPERTS, MOE_FF, HIDDEN),
            const(N_EXPERTS, HIDDEN, MOE_FF),
            const(SHARED_FF, HIDDEN),
            const(HIDDEN, SHARED_FF),
        ],
        out_specs=pl.BlockSpec((BT, HIDDEN), lambda i: (i, 0)),
        out_shape=jax.ShapeDtypeStruct((t, HIDDEN), jnp.float32),
    )(hs2, router_w, e_bias2, expert_up, expert_down, shared_up, shared_down)


def kernel(hidden_states, router_w, e_bias, expert_up, expert_down, shared_up, shared_down):
    orig_shape = hidden_states.shape
    hs2 = hidden_states.reshape(-1, orig_shape[-1])
    out = _moe(hs2, router_w, e_bias.reshape(1, N_EXPERTS), expert_up,
               expert_down, shared_up, shared_down)
    return out.reshape(orig_shape).astype(hidden_states.dtype)
